# baseline (speedup 1.0000x reference)
import jax
import jax.numpy as jnp
from jax import lax
from jax.experimental import pallas as pl
from jax.experimental.pallas import tpu as pltpu
from jax.experimental.pallas import tpu_sc as plsc

D = 1001000
LAT = 64
C = 520
CP = 528
G = CP // 16
NCHUNK = D // C
NC = 2
NS = 16
NW = NC * NS


def _sc_body(z_hbm, theta_hbm, basis_hbm, out_hbm, z_v, basis_v, acc_v):
    wid = lax.axis_index("s") * NC + lax.axis_index("c")
    n_mine = NCHUNK // NW + jnp.where(wid < NCHUNK % NW, 1, 0)

    pltpu.sync_copy(z_hbm, z_v)
    biota = lax.iota(jnp.int32, 16) * LAT

    def do_chunk(i, carry):
        c = wid + i * NW
        row0 = pl.multiple_of(c * C, 8)
        flat0 = pl.multiple_of(c * (C * LAT), 64)
        pltpu.sync_copy(basis_hbm.at[pl.ds(flat0, C * LAT)],
                        basis_v.at[pl.ds(0, C * LAT)])
        pltpu.sync_copy(theta_hbm.at[pl.ds(row0, C)], acc_v.at[pl.ds(0, C)])

        def do_k(k, carry_k):
            zb = plsc.load_gather(z_v, [jnp.zeros((16,), jnp.int32) + k])
            for g in range(G):
                idx = biota + (g * 16 * LAT + k)
                col = plsc.load_gather(basis_v, [idx])
                plsc.addupdate(acc_v.at[pl.ds(g * 16, 16)], col * zb)
            return carry_k

        lax.fori_loop(0, LAT, do_k, 0)
        pltpu.sync_copy(acc_v.at[pl.ds(0, C)], out_hbm.at[pl.ds(row0, C)])
        return carry

    lax.fori_loop(0, n_mine, do_chunk, 0)


@jax.jit
def _projection(z, theta_base, basis_flat):
    mesh = plsc.VectorSubcoreMesh(core_axis_name="c", subcore_axis_name="s",
                                  num_cores=NC, num_subcores=NS)
    return pl.kernel(
        _sc_body,
        out_type=jax.ShapeDtypeStruct((D,), jnp.float32),
        mesh=mesh,
        scratch_types=[
            pltpu.VMEM((LAT,), jnp.float32),
            pltpu.VMEM((CP * LAT,), jnp.float32),
            pltpu.VMEM((CP,), jnp.float32),
        ],
        compiler_params=pltpu.CompilerParams(needs_layout_passes=False),
    )(z, theta_base, basis_flat)


def kernel(z, theta_base, basis_matrix):
    return _projection(z, theta_base, basis_matrix.reshape(-1))


# speedup vs baseline: 1.8007x; 1.8007x over previous
import jax
import jax.numpy as jnp
from jax import lax
from jax.experimental import pallas as pl
from jax.experimental.pallas import tpu as pltpu
from jax.experimental.pallas import tpu_sc as plsc

D = 1001000
LAT = 64
C = 440
CP = 448
G = CP // 16
NCHUNK = D // C
NC = 2
NS = 16
NW = NC * NS
PASSES = ((0, 14), (14, 28))


def _sc_body(z_hbm, theta_hbm, basis_hbm, out_hbm,
             z_v, b0_v, b1_v, a0_v, a1_v, bsem0, bsem1, osem0, osem1):
    wid = lax.axis_index("s") * NC + lax.axis_index("c")
    n_mine = NCHUNK // NW + jnp.where(wid < NCHUNK % NW, 1, 0)

    pltpu.sync_copy(z_hbm, z_v)
    iota16 = lax.iota(jnp.int32, 16)

    bufs = ((b0_v, a0_v, bsem0, osem0), (b1_v, a1_v, bsem1, osem1))

    def chunk_row0(j):
        return pl.multiple_of((wid + j * NW) * C, 8)

    def start_in(j, b):
        row0 = chunk_row0(j)
        bv, av, bsem, _ = bufs[b]
        pltpu.async_copy(basis_hbm.at[pl.ds(row0, C), :],
                         bv.at[pl.ds(0, C), :], bsem)
        pltpu.async_copy(theta_hbm.at[pl.ds(row0, C)], av.at[pl.ds(0, C)],
                         bsem)

    def wait_in(b):
        bv, av, bsem, _ = bufs[b]
        pltpu.make_async_copy(basis_hbm.at[pl.ds(0, C), :],
                              bv.at[pl.ds(0, C), :], bsem).wait()
        pltpu.make_async_copy(theta_hbm.at[pl.ds(0, C)], av.at[pl.ds(0, C)],
                              bsem).wait()

    def start_out(j, b):
        row0 = chunk_row0(j)
        _, av, _, osem = bufs[b]
        pltpu.async_copy(av.at[pl.ds(0, C)], out_hbm.at[pl.ds(row0, C)], osem)

    def wait_out(b):
        _, av, _, osem = bufs[b]
        pltpu.make_async_copy(av.at[pl.ds(0, C)], out_hbm.at[pl.ds(0, C)],
                              osem).wait()

    def compute(b):
        bv, av, _, _ = bufs[b]
        for lo, hi in PASSES:
            ng = hi - lo
            accs0 = tuple(av[pl.ds(g * 16, 16)] for g in range(lo, hi))
            rows = tuple(iota16 + (g * 16) for g in range(lo, hi))

            def do_k(k, accs):
                kv = jnp.zeros((16,), jnp.int32) + k
                zb = plsc.load_gather(z_v, [kv])
                new = []
                for t in range(ng):
                    col = plsc.load_gather(bv, [rows[t], kv])
                    new.append(accs[t] + col * zb)
                return tuple(new)

            accs = lax.fori_loop(0, LAT, do_k, accs0, unroll=2)
            for t, g in enumerate(range(lo, hi)):
                av[pl.ds(g * 16, 16)] = accs[t]

    @pl.when(n_mine > 0)
    def _prologue():
        start_in(0, 0)

    def pair_body(p, carry):
        for b in (0, 1):
            j = 2 * p + b

            @pl.when(j < n_mine)
            def _step():
                @pl.when(j + 1 < n_mine)
                def _prefetch():
                    @pl.when(j >= 1)
                    def _drain():
                        wait_out(1 - b)
                    start_in(j + 1, 1 - b)

                wait_in(b)
                compute(b)
                start_out(j, b)
        return carry

    lax.fori_loop(0, (n_mine + 1) // 2, pair_body, 0)

    last = n_mine - 1

    @pl.when(n_mine > 0)
    def _final_drain():
        @pl.when(last % 2 == 0)
        def _d0():
            wait_out(0)

        @pl.when(last % 2 == 1)
        def _d1():
            wait_out(1)


@jax.jit
def _projection(z, theta_base, basis_matrix):
    mesh = plsc.VectorSubcoreMesh(core_axis_name="c", subcore_axis_name="s",
                                  num_cores=NC, num_subcores=NS)
    return pl.kernel(
        _sc_body,
        out_type=jax.ShapeDtypeStruct((D,), jnp.float32),
        mesh=mesh,
        scratch_types=[
            pltpu.VMEM((LAT,), jnp.float32),
            pltpu.VMEM((CP, LAT), jnp.float32),
            pltpu.VMEM((CP, LAT), jnp.float32),
            pltpu.VMEM((CP,), jnp.float32),
            pltpu.VMEM((CP,), jnp.float32),
            pltpu.SemaphoreType.DMA,
            pltpu.SemaphoreType.DMA,
            pltpu.SemaphoreType.DMA,
            pltpu.SemaphoreType.DMA,
        ],
        compiler_params=pltpu.CompilerParams(needs_layout_passes=False),
    )(z, theta_base, basis_matrix)


def kernel(z, theta_base, basis_matrix):
    return _projection(z, theta_base, basis_matrix)


# speedup vs baseline: 18.0252x; 10.0100x over previous
import jax
import jax.numpy as jnp
from jax import lax
from jax.experimental import pallas as pl
from jax.experimental.pallas import tpu as pltpu
from jax.experimental.pallas import tpu_sc as plsc

D = 1001000
LAT = 64
W = 384
NG = W // 16
DPAD = 1001088
NCHUNK = DPAD // W
NC = 2
NS = 16
NW = NC * NS


def _sc_body(zt_hbm, theta_hbm, bT_hbm, out_hbm,
             zt_v, b0_v, b1_v, a0_v, a1_v, bsem0, bsem1, osem0, osem1):
    wid = lax.axis_index("s") * NC + lax.axis_index("c")
    n_mine = NCHUNK // NW + jnp.where(wid < NCHUNK % NW, 1, 0)

    pltpu.sync_copy(zt_hbm, zt_v)

    bufs = ((b0_v, a0_v, bsem0, osem0), (b1_v, a1_v, bsem1, osem1))

    def chunk_col0(j):
        return pl.multiple_of((wid + j * NW) * W, 128)

    def start_in(j, b):
        col0 = chunk_col0(j)
        bv, av, bsem, _ = bufs[b]
        pltpu.async_copy(bT_hbm.at[:, pl.ds(col0, W)], bv, bsem)
        pltpu.async_copy(theta_hbm.at[pl.ds(col0, W)], av, bsem)

    def wait_in(b):
        bv, av, bsem, _ = bufs[b]
        pltpu.make_async_copy(bT_hbm.at[:, pl.ds(0, W)], bv, bsem).wait()
        pltpu.make_async_copy(theta_hbm.at[pl.ds(0, W)], av, bsem).wait()

    def start_out(j, b):
        col0 = chunk_col0(j)
        _, av, _, osem = bufs[b]
        pltpu.async_copy(av, out_hbm.at[pl.ds(col0, W)], osem)

    def wait_out(b):
        _, av, _, osem = bufs[b]
        pltpu.make_async_copy(av, out_hbm.at[pl.ds(0, W)], osem).wait()

    def compute(b):
        bv, av, _, _ = bufs[b]
        accs0 = tuple(av[pl.ds(g * 16, 16)] for g in range(NG))

        def do_k(k, accs):
            zb = zt_v[pl.ds(pl.multiple_of(k * 16, 16), 16)]
            return tuple(accs[g] + bv[k, pl.ds(g * 16, 16)] * zb
                         for g in range(NG))

        accs = lax.fori_loop(0, LAT, do_k, accs0, unroll=2)
        for g in range(NG):
            av[pl.ds(g * 16, 16)] = accs[g]

    @pl.when(n_mine > 0)
    def _prologue():
        start_in(0, 0)

    def pair_body(p, carry):
        for b in (0, 1):
            j = 2 * p + b

            @pl.when(j < n_mine)
            def _step():
                @pl.when(j + 1 < n_mine)
                def _prefetch():
                    @pl.when(j >= 1)
                    def _drain():
                        wait_out(1 - b)
                    start_in(j + 1, 1 - b)

                wait_in(b)
                compute(b)
                start_out(j, b)
        return carry

    lax.fori_loop(0, (n_mine + 1) // 2, pair_body, 0)

    last = n_mine - 1

    @pl.when(n_mine > 0)
    def _final_drain():
        @pl.when(last % 2 == 0)
        def _d0():
            wait_out(0)

        @pl.when(last % 2 == 1)
        def _d1():
            wait_out(1)


@jax.jit
def _projection(zt, theta_base, bT):
    mesh = plsc.VectorSubcoreMesh(core_axis_name="c", subcore_axis_name="s",
                                  num_cores=NC, num_subcores=NS)
    return pl.kernel(
        _sc_body,
        out_type=jax.ShapeDtypeStruct((D,), jnp.float32),
        mesh=mesh,
        scratch_types=[
            pltpu.VMEM((LAT * 16,), jnp.float32),
            pltpu.VMEM((LAT, W), jnp.float32),
            pltpu.VMEM((LAT, W), jnp.float32),
            pltpu.VMEM((W,), jnp.float32),
            pltpu.VMEM((W,), jnp.float32),
            pltpu.SemaphoreType.DMA,
            pltpu.SemaphoreType.DMA,
            pltpu.SemaphoreType.DMA,
            pltpu.SemaphoreType.DMA,
        ],
        compiler_params=pltpu.CompilerParams(needs_layout_passes=False),
    )(zt, theta_base, bT)


def kernel(z, theta_base, basis_matrix):
    zt = jnp.broadcast_to(z[:, None], (LAT, 16)).reshape(-1)
    return _projection(zt, theta_base, basis_matrix.T)
